# Initial kernel scaffold; baseline (speedup 1.0000x reference)
import jax
import jax.numpy as jnp
from jax.experimental import pallas as pl


def kernel(t, x, means, vars_, y_means, y_vars):
    raise NotImplementedError("write your pallas kernel here")


# speedup vs baseline: 1.0310x; 1.0310x over previous
import jax
import jax.numpy as jnp
from jax.experimental import pallas as pl
from jax.experimental.pallas import tpu as pltpu

N_CLUSTERS = 8192
N_F = 2048
N_T = 512
K_BLK = 512
N_BLOCKS = N_CLUSTERS // K_BLK


def _em_kernel(x_ref, means_ref, vars_ref, y_means_ref, out_ref,
               best_val, best_idx, sem):
    k = pl.program_id(0)

    @pl.when(k == 0)
    def _init():
        best_val[0] = jnp.inf

    x = x_ref[...]
    m = means_ref[...]
    v = vars_ref[...]
    d = m - x
    s = jnp.sum(d * d / v + jnp.log(v), axis=1, keepdims=True)

    bmin = jnp.min(s)
    idx2 = jax.lax.broadcasted_iota(jnp.int32, (K_BLK, 1), 0)
    bidx = jnp.min(jnp.where(s == bmin, idx2, K_BLK))

    @pl.when(bmin < best_val[0])
    def _update():
        best_val[0] = bmin
        best_idx[0] = k * K_BLK + bidx

    @pl.when(k == N_BLOCKS - 1)
    def _gather():
        i = best_idx[0]
        cp = pltpu.make_async_copy(
            y_means_ref.at[pl.ds(i, 1), :], out_ref, sem)
        cp.start()
        cp.wait()


def kernel(t, x, means, vars_, y_means, y_vars):
    out = pl.pallas_call(
        _em_kernel,
        grid=(N_BLOCKS,),
        in_specs=[
            pl.BlockSpec((1, N_F), lambda k: (0, 0)),
            pl.BlockSpec((K_BLK, N_F), lambda k: (k, 0)),
            pl.BlockSpec((K_BLK, N_F), lambda k: (k, 0)),
            pl.BlockSpec(memory_space=pl.ANY),
        ],
        out_specs=pl.BlockSpec((1, N_T), lambda k: (0, 0)),
        out_shape=jax.ShapeDtypeStruct((1, N_T), jnp.float32),
        scratch_shapes=[
            pltpu.SMEM((1,), jnp.float32),
            pltpu.SMEM((1,), jnp.int32),
            pltpu.SemaphoreType.DMA,
        ],
        compiler_params=pltpu.CompilerParams(
            dimension_semantics=("arbitrary",),
        ),
    )(x.reshape(1, N_F), means, vars_, y_means)
    return out.reshape(N_T)
